# baseline (speedup 1.0000x reference)
import jax
import jax.numpy as jnp
from jax.experimental import pallas as pl
from jax.experimental.pallas import tpu as pltpu

_B, _D, _N = 32, 64, 576
_K = 1024


_BB = 8
_KC = 128


def _vq_kernel(x_ref, emb_ref, out_ref, idx_ref):
    emb = emb_ref[...]
    e2h = 0.5 * jnp.sum(emb * emb, axis=1, keepdims=True)
    for b in range(_BB):
        x = x_ref[b]
        v = None
        for c in range(_K // _KC):
            embc = emb[c * _KC:(c + 1) * _KC]
            sc = jax.lax.dot_general(
                embc, x, (((1,), (0,)), ((), ())),
                preferred_element_type=jnp.float32)
            d2c = e2h[c * _KC:(c + 1) * _KC] - sc
            vc = jnp.min(d2c, axis=0)
            ic = jnp.argmin(d2c, axis=0) + c * _KC
            if v is None:
                v, idx = vc, ic
            else:
                better = vc < v
                idx = jnp.where(better, ic, idx)
                v = jnp.where(better, vc, v)
        idx_ref[b, 0, :] = idx
        onehot = (jax.lax.broadcasted_iota(jnp.int32, (_K, _N), 0)
                  == idx[None, :]).astype(jnp.float32)
        out = jax.lax.dot_general(
            emb, onehot, (((0,), (0,)), ((), ())),
            preferred_element_type=jnp.float32)
        out_ref[b] = out


def kernel(x_in, codebook):
    out, idx = pl.pallas_call(
        _vq_kernel,
        grid=(_B // _BB,),
        in_specs=[
            pl.BlockSpec((_BB, _D, _N), lambda b: (b, 0, 0)),
            pl.BlockSpec((_K, _D), lambda b: (0, 0)),
        ],
        out_specs=[
            pl.BlockSpec((_BB, _D, _N), lambda b: (b, 0, 0)),
            pl.BlockSpec((_BB, 1, _N), lambda b: (b, 0, 0)),
        ],
        out_shape=[
            jax.ShapeDtypeStruct((_B, _D, _N), jnp.float32),
            jax.ShapeDtypeStruct((_B, 1, _N), jnp.int32),
        ],
        compiler_params=pltpu.CompilerParams(
            dimension_semantics=("parallel",)),
    )(x_in, codebook)
    return out, jnp.reshape(idx, (_B, _N, 1))


# speedup vs baseline: 1.2917x; 1.2917x over previous
import jax
import jax.numpy as jnp
from jax.experimental import pallas as pl
from jax.experimental.pallas import tpu as pltpu

_B, _D, _N = 32, 64, 576
_K = 1024


_BB = 8


def _vq_kernel(x_ref, emb_ref, out_ref, idx_ref):
    emb = emb_ref[...]
    e2h = 0.5 * jnp.sum(emb * emb, axis=1, keepdims=True)
    for b in range(_BB):
        x = x_ref[b]
        s = jax.lax.dot_general(
            emb, x, (((1,), (0,)), ((), ())),
            preferred_element_type=jnp.float32)
        d2 = e2h - s
        idx = jnp.argmin(d2, axis=0)
        idx_ref[b, 0, :] = idx
        onehot = (jax.lax.broadcasted_iota(jnp.int32, (_K, _N), 0)
                  == idx[None, :]).astype(jnp.float32)
        out = jax.lax.dot_general(
            emb, onehot, (((0,), (0,)), ((), ())),
            preferred_element_type=jnp.float32)
        out_ref[b] = out


def kernel(x_in, codebook):
    out, idx = pl.pallas_call(
        _vq_kernel,
        grid=(_B // _BB,),
        in_specs=[
            pl.BlockSpec((_BB, _D, _N), lambda b: (b, 0, 0)),
            pl.BlockSpec((_K, _D), lambda b: (0, 0)),
        ],
        out_specs=[
            pl.BlockSpec((_BB, _D, _N), lambda b: (b, 0, 0)),
            pl.BlockSpec((_BB, 1, _N), lambda b: (b, 0, 0)),
        ],
        out_shape=[
            jax.ShapeDtypeStruct((_B, _D, _N), jnp.float32),
            jax.ShapeDtypeStruct((_B, 1, _N), jnp.int32),
        ],
        compiler_params=pltpu.CompilerParams(
            dimension_semantics=("arbitrary",)),
    )(x_in, codebook)
    return out, jnp.reshape(idx, (_B, _N, 1))
